# baseline (speedup 1.0000x reference)
import functools

import jax
import jax.numpy as jnp
from jax import lax
from jax.experimental import pallas as pl
from jax.experimental.pallas import tpu as pltpu
from jax.experimental.pallas import tpu_sc as plsc

N = 10000
E = 320000
DIN = 512
D = 128

NC = 2
NS = 16
NW = NC * NS
RNG = 320
NPAD = NW * RNG
ACC_ROWS = 328
DUMP = 327
WAVE = 128
BCAP = 14592

E_PER_W = E // NW
SUM_WAVE = 80
SUM_NW = E_PER_W // SUM_WAVE
SPAD = 10240
SSTRIPE = SPAD // NS
SCHUNK = SSTRIPE // 2

BLK = 1000
GRID = N // BLK

_mesh = functools.partial(
    plsc.VectorSubcoreMesh, core_axis_name="c", subcore_axis_name="s")


def _lane():
    return lax.iota(jnp.int32, 16)


def _rup128(x):
    return (x + 127) & ~jnp.int32(127)


def _bin_body(src_hbm, dst_hbm, binned_out, counts_out, histn_out,
              sv_v, dv_v, bv_v, pk_v, hist, cur, buf, histn):
    c = lax.axis_index("c")
    s = lax.axis_index("s")
    wid = s * NC + c
    base_e = wid * E_PER_W

    pltpu.sync_copy(src_hbm.at[pl.ds(base_e, E_PER_W)],
                    sv_v.at[pl.ds(0, E_PER_W)])
    pltpu.sync_copy(dst_hbm.at[pl.ds(base_e, E_PER_W)],
                    dv_v.at[pl.ds(0, E_PER_W)])

    zi = jnp.zeros((16,), jnp.int32)
    one0 = jnp.where(_lane() == 0, 1, 0)
    for j in range(48 // 16):
        hist[pl.ds(j * 16, 16)] = zi

    def zh(i, _):
        histn[pl.ds(i * 16, 16)] = zi
        return 0
    lax.fori_loop(0, NPAD // 16, zh, 0)

    def prep(i, _):
        sl = pl.ds(i * 16, 16)
        d = dv_v[sl]
        b = lax.shift_right_logical(d * 6554, 21)
        bv_v[sl] = b
        pk_v[sl] = lax.shift_left(sv_v[sl], 9) | (d - b * RNG)
        return 0
    lax.fori_loop(0, E_PER_W // 16, prep, 0)

    def h1(e, _):
        d = dv_v[pl.ds(e, 16)][0]
        bsl = pl.ds(lax.shift_right_logical(d * 6554, 21), 16)
        hist[bsl] = hist[bsl] + one0
        dsl = pl.ds(d, 16)
        histn[dsl] = histn[dsl] + one0
        return 0
    lax.fori_loop(0, E_PER_W, h1, 0)

    def st(b, off):
        cur[pl.ds(b, 16)] = zi + off
        return off + _rup128(hist[pl.ds(b, 16)][0] + 16)
    lax.fori_loop(0, NW, st, 0)

    dumpv = jnp.full((16,), DUMP, jnp.int32)

    def zb(i, _):
        buf[pl.ds(i * 16, 16)] = dumpv
        return 0
    lax.fori_loop(0, BCAP // 16, zb, 0)

    lane0 = _lane() == 0

    def h2(e, _):
        b = bv_v[pl.ds(e, 16)][0]
        bsl = pl.ds(b, 16)
        cb = cur[bsl]
        o = cb[0]
        buf[pl.ds(o, 16)] = jnp.where(lane0, pk_v[pl.ds(e, 16)], dumpv)
        cur[bsl] = cb + one0
        return 0
    lax.fori_loop(0, E_PER_W, h2, 0)

    pltpu.sync_copy(buf.at[pl.ds(0, BCAP)],
                    binned_out.at[pl.ds(wid * BCAP, BCAP)])
    pltpu.sync_copy(hist.at[pl.ds(0, NW)],
                    counts_out.at[pl.ds(wid * NW, NW)])
    pltpu.sync_copy(histn.at[pl.ds(0, NPAD)],
                    histn_out.at[pl.ds(wid * NPAD, NPAD)])


def _bin_edges(src, dst):
    return pl.kernel(
        _bin_body,
        out_type=(
            jax.ShapeDtypeStruct((NW * BCAP,), jnp.int32),
            jax.ShapeDtypeStruct((NW * NW,), jnp.int32),
            jax.ShapeDtypeStruct((NW * NPAD,), jnp.int32),
        ),
        mesh=_mesh(),
        scratch_types=[
            pltpu.VMEM((E_PER_W + 16,), jnp.int32),
            pltpu.VMEM((E_PER_W + 16,), jnp.int32),
            pltpu.VMEM((E_PER_W + 16,), jnp.int32),
            pltpu.VMEM((E_PER_W + 16,), jnp.int32),
            pltpu.VMEM((48,), jnp.int32),
            pltpu.VMEM((48,), jnp.int32),
            pltpu.VMEM((BCAP + 16,), jnp.int32),
            pltpu.VMEM((NPAD + 16,), jnp.int32),
        ],
    )(src, dst)


WCAP = 2656


def _segmax_body(binned_hbm, counts_hbm, m_hbm, agg_out,
                 acc, cnt_v, woff,
                 pkb0, pkb1, widx0, widx1, rows0, rows1,
                 psem0, psem1, gsem0, gsem1):
    c = lax.axis_index("c")
    s = lax.axis_index("s")
    wid = s * NC + c

    zf = jnp.zeros((16,), jnp.float32)
    zi = jnp.zeros((16,), jnp.int32)

    def zero_row(r, _):
        for j in range(8):
            acc[r, pl.ds(j * 16, 16)] = zf
        return 0
    lax.fori_loop(0, ACC_ROWS, zero_row, 0)

    pltpu.sync_copy(counts_hbm, cnt_v.at[pl.ds(0, NW * NW)])

    def seg(t, nwt):
        def st(b, off):
            return off + _rup128(cnt_v[pl.ds(t * NW + b, 16)][0] + 16)
        offw = lax.fori_loop(0, wid, st, 0)
        cnt = cnt_v[pl.ds(t * NW + wid, 16)][0]
        nwv = (cnt + (WAVE - 1)) // WAVE

        def put(v, _):
            woff[pl.ds(nwt + v, 16)] = zi + (t * BCAP + offw + v * WAVE)
            return 0
        lax.fori_loop(0, nwv, put, 0)
        return nwt + nwv
    nw_tot = lax.fori_loop(0, NW, seg, 0)
    for k in range(3):
        woff[pl.ds(nw_tot + k * 16, 16)] = zi

    def fire_pk(w, dst, sem):
        off = pl.multiple_of(woff[pl.ds(w, 16)][0], WAVE)
        pltpu.async_copy(binned_hbm.at[pl.ds(off, WAVE)],
                         dst.at[pl.ds(0, WAVE)], sem)

    def wait_pk(dst, sem):
        pltpu.make_async_copy(binned_hbm.at[pl.ds(0, WAVE)],
                              dst.at[pl.ds(0, WAVE)], sem).wait()

    def unpack(pkb, widx):
        for j in range(WAVE // 16):
            sl = pl.ds(j * 16, 16)
            widx[sl] = lax.shift_right_logical(pkb[sl], 9)

    def accum(pkb, rows):
        def edge(e, _):
            dl = pkb[pl.ds(e, 16)][0] & 511
            for j in range(8):
                sl2 = pl.ds(j * 16, 16)
                acc[dl, sl2] = jnp.maximum(acc[dl, sl2], rows[e, sl2])
            return 0
        lax.fori_loop(0, WAVE, edge, 0)

    fire_pk(0, pkb0, psem0)
    npair = (nw_tot + 1) // 2

    def pair(p, _):
        w = p * 2
        wait_pk(pkb0, psem0)
        unpack(pkb0, widx0)
        pltpu.async_copy(m_hbm.at[widx0], rows0, gsem0)
        fire_pk(w + 1, pkb1, psem1)
        wait_pk(pkb1, psem1)
        unpack(pkb1, widx1)
        pltpu.make_async_copy(m_hbm.at[widx0], rows0, gsem0).wait()
        pltpu.async_copy(m_hbm.at[widx1], rows1, gsem1)
        accum(pkb0, rows0)
        pltpu.make_async_copy(m_hbm.at[widx1], rows1, gsem1).wait()
        accum(pkb1, rows1)
        fire_pk(w + 2, pkb0, psem0)
        return 0
    lax.fori_loop(0, npair, pair, 0)
    wait_pk(pkb0, psem0)

    pltpu.sync_copy(acc.at[pl.ds(0, RNG)], agg_out.at[wid])


def _segmax(binned, counts, m):
    return pl.kernel(
        _segmax_body,
        out_type=jax.ShapeDtypeStruct((NW, RNG, D), jnp.float32),
        mesh=_mesh(),
        scratch_types=[
            pltpu.VMEM((ACC_ROWS, D), jnp.float32),
            pltpu.VMEM((NW * NW + 16,), jnp.int32),
            pltpu.VMEM((WCAP + 64,), jnp.int32),
            pltpu.VMEM((WAVE,), jnp.int32),
            pltpu.VMEM((WAVE,), jnp.int32),
            pltpu.VMEM((WAVE,), jnp.int32),
            pltpu.VMEM((WAVE,), jnp.int32),
            pltpu.VMEM((WAVE, D), jnp.float32),
            pltpu.VMEM((WAVE, D), jnp.float32),
            pltpu.SemaphoreType.DMA,
            pltpu.SemaphoreType.DMA,
            pltpu.SemaphoreType.DMA,
            pltpu.SemaphoreType.DMA,
        ],
    )(binned, counts, m)


def _segsum_body(src_hbm, dst_hbm, h1_hbm, out_hbm,
                 shared, buf, sidx, didx, sem):
    c = lax.axis_index("c")
    s = lax.axis_index("s")
    wid = s * NC + c
    base_e = wid * E_PER_W

    zf = jnp.zeros((16,), jnp.float32)

    def zero_row(r, _):
        for j in range(8):
            buf[r, pl.ds(j * 16, 16)] = zf
        return 0
    lax.fori_loop(0, SCHUNK, zero_row, 0)
    for k in range(2):
        pltpu.sync_copy(buf,
                        shared.at[pl.ds(s * SSTRIPE + k * SCHUNK, SCHUNK)])
    plsc.subcore_barrier()

    def wave(w, _):
        e0 = base_e + w * SUM_WAVE
        pltpu.sync_copy(src_hbm.at[pl.ds(e0, SUM_WAVE)], sidx)
        pltpu.sync_copy(dst_hbm.at[pl.ds(e0, SUM_WAVE)], didx)
        pltpu.async_copy(h1_hbm.at[sidx], buf.at[pl.ds(0, SUM_WAVE)],
                         sem).wait()
        pltpu.sync_copy(buf.at[pl.ds(0, SUM_WAVE)], shared.at[didx],
                        add=True)
        return 0
    lax.fori_loop(0, SUM_NW, wave, 0)

    plsc.subcore_barrier()

    for k in range(2):
        r0 = s * SSTRIPE + k * SCHUNK
        pltpu.sync_copy(shared.at[pl.ds(r0, SCHUNK)], buf)
        pltpu.sync_copy(buf, out_hbm.at[c].at[pl.ds(r0, SCHUNK)])


def _segsum(src, dst, h1):
    return pl.kernel(
        _segsum_body,
        out_type=jax.ShapeDtypeStruct((NC, SPAD, D), jnp.float32),
        mesh=_mesh(),
        scratch_types=[
            pltpu.VMEM_SHARED((SPAD, D), jnp.float32),
            pltpu.VMEM((SCHUNK, D), jnp.float32),
            pltpu.VMEM((SUM_WAVE,), jnp.int32),
            pltpu.VMEM((SUM_WAVE,), jnp.int32),
            pltpu.SemaphoreType.DMA,
        ],
    )(src, dst, h1)


def _edge_body(src_hbm, dst_hbm, a_hbm, b_hbm, out_hbm,
               a_v, b_v, s_v, d_v, o_v):
    c = lax.axis_index("c")
    s = lax.axis_index("s")
    wid = s * NC + c
    base_e = wid * E_PER_W

    pltpu.sync_copy(a_hbm, a_v.at[pl.ds(0, N)])
    pltpu.sync_copy(b_hbm, b_v.at[pl.ds(0, N)])
    pltpu.sync_copy(src_hbm.at[pl.ds(base_e, E_PER_W)],
                    s_v.at[pl.ds(0, E_PER_W)])
    pltpu.sync_copy(dst_hbm.at[pl.ds(base_e, E_PER_W)],
                    d_v.at[pl.ds(0, E_PER_W)])

    def lp(e, _):
        si = s_v[pl.ds(e, 16)][0]
        di = d_v[pl.ds(e, 16)][0]
        va = a_v[pl.ds(si, 16)][0]
        vb = b_v[pl.ds(di, 16)][0]
        o_v[pl.ds(e, 16)] = jnp.zeros((16,), jnp.float32) + (va + vb)
        return 0
    lax.fori_loop(0, E_PER_W, lp, 0)

    pltpu.sync_copy(o_v.at[pl.ds(0, E_PER_W)],
                    out_hbm.at[pl.ds(base_e, E_PER_W)])


def _edge_scores(src, dst, a, b):
    return pl.kernel(
        _edge_body,
        out_type=jax.ShapeDtypeStruct((E,), jnp.float32),
        mesh=_mesh(),
        scratch_types=[
            pltpu.VMEM((N + 16,), jnp.float32),
            pltpu.VMEM((N + 16,), jnp.float32),
            pltpu.VMEM((E_PER_W + 16,), jnp.int32),
            pltpu.VMEM((E_PER_W + 16,), jnp.int32),
            pltpu.VMEM((E_PER_W + 16,), jnp.float32),
        ],
    )(src, dst, a, b)


def _tc1_body(x_ref, wr_ref, br_ref, wp_ref, bp_ref, h_ref, m_ref):
    h = jnp.dot(x_ref[...], wr_ref[...],
                preferred_element_type=jnp.float32) + br_ref[...]
    h_ref[...] = h
    m = jnp.dot(h, wp_ref[...], preferred_element_type=jnp.float32)
    m_ref[...] = jnp.maximum(m + bp_ref[...], 0.0)


def _tc1(x, W_red, b_red, W_pool, b_pool):
    return pl.pallas_call(
        _tc1_body,
        grid=(GRID,),
        in_specs=[
            pl.BlockSpec((BLK, DIN), lambda i: (i, 0)),
            pl.BlockSpec((DIN, D), lambda i: (0, 0)),
            pl.BlockSpec((1, D), lambda i: (0, 0)),
            pl.BlockSpec((D, D), lambda i: (0, 0)),
            pl.BlockSpec((1, D), lambda i: (0, 0)),
        ],
        out_specs=[
            pl.BlockSpec((BLK, D), lambda i: (i, 0)),
            pl.BlockSpec((BLK, D), lambda i: (i, 0)),
        ],
        out_shape=[
            jax.ShapeDtypeStruct((N, D), jnp.float32),
            jax.ShapeDtypeStruct((N, D), jnp.float32),
        ],
    )(x, W_red, b_red.reshape(1, D), W_pool, b_pool.reshape(1, D))


def _tc2_body(h_ref, agg_ref, ws_ref, wn_ref, b_ref, hist_ref,
              h1_ref, deg_ref):
    r = (jnp.dot(h_ref[...], ws_ref[...], preferred_element_type=jnp.float32)
         + jnp.dot(agg_ref[...], wn_ref[...],
                   preferred_element_type=jnp.float32)
         + b_ref[...])
    n = jnp.sqrt(jnp.sum(r * r, axis=-1, keepdims=True))
    r = r / jnp.maximum(n, 1e-12)
    h1_ref[...] = jnp.maximum(r, 0.0)
    deg_ref[...] = jnp.sum(hist_ref[...].astype(jnp.float32), axis=1,
                           keepdims=True)


def _tc2(h, agg, W_self1, W_neigh1, b1, hists):
    return pl.pallas_call(
        _tc2_body,
        grid=(GRID,),
        in_specs=[
            pl.BlockSpec((BLK, D), lambda i: (i, 0)),
            pl.BlockSpec((BLK, D), lambda i: (i, 0)),
            pl.BlockSpec((D, D), lambda i: (0, 0)),
            pl.BlockSpec((D, D), lambda i: (0, 0)),
            pl.BlockSpec((1, D), lambda i: (0, 0)),
            pl.BlockSpec((BLK, NW), lambda i: (i, 0)),
        ],
        out_specs=[
            pl.BlockSpec((BLK, D), lambda i: (i, 0)),
            pl.BlockSpec((BLK, 1), lambda i: (i, 0)),
        ],
        out_shape=[
            jax.ShapeDtypeStruct((N, D), jnp.float32),
            jax.ShapeDtypeStruct((N, 1), jnp.float32),
        ],
    )(h, agg, W_self1, W_neigh1, b1.reshape(1, D), hists)


def _tc3_body(h1_ref, s0_ref, s1_ref, deg_ref, ws_ref, wn_ref, b_ref,
              we_ref, be_ref, ab_ref):
    mean = (s0_ref[...] + s1_ref[...]) / jnp.maximum(deg_ref[...], 1.0)
    h2 = (jnp.dot(h1_ref[...], ws_ref[...],
                  preferred_element_type=jnp.float32)
          + jnp.dot(mean, wn_ref[...], preferred_element_type=jnp.float32)
          + b_ref[...])
    h2 = jnp.maximum(h2, 0.0)
    ab = jnp.dot(h2, we_ref[...], preferred_element_type=jnp.float32)
    ab_ref[...] = ab + be_ref[...]


def _tc3(h1, s0, s1, deg, W_self2, W_neigh2, b2, we_p, be_p):
    return pl.pallas_call(
        _tc3_body,
        grid=(GRID,),
        in_specs=[
            pl.BlockSpec((BLK, D), lambda i: (i, 0)),
            pl.BlockSpec((BLK, D), lambda i: (i, 0)),
            pl.BlockSpec((BLK, D), lambda i: (i, 0)),
            pl.BlockSpec((BLK, 1), lambda i: (i, 0)),
            pl.BlockSpec((D, D), lambda i: (0, 0)),
            pl.BlockSpec((D, D), lambda i: (0, 0)),
            pl.BlockSpec((1, D), lambda i: (0, 0)),
            pl.BlockSpec((D, 8), lambda i: (0, 0)),
            pl.BlockSpec((1, 8), lambda i: (0, 0)),
        ],
        out_specs=pl.BlockSpec((BLK, 8), lambda i: (i, 0)),
        out_shape=jax.ShapeDtypeStruct((N, 8), jnp.float32),
    )(h1, s0, s1, deg, W_self2, W_neigh2, b2.reshape(1, D), we_p, be_p)


def kernel(x, edge_index, W_red, b_red, W_pool, b_pool, W_self1, W_neigh1,
           b1, W_self2, W_neigh2, b2, W_e, b_e):
    src = edge_index[0]
    dst = edge_index[1]

    h, m = _tc1(x, W_red, b_red, W_pool, b_pool)

    binned, counts, histn = _bin_edges(src, dst)
    agg_t = _segmax(binned, counts, m)
    agg = agg_t.reshape(NPAD, D)[:N]
    hists = histn.reshape(NW, NPAD)[:, :N].T

    h1, deg = _tc2(h, agg, W_self1, W_neigh1, b1, hists)

    s_part = _segsum(src, dst, h1)
    s_full = s_part[:, :N, :]

    we_p = jnp.zeros((D, 8), jnp.float32)
    we_p = we_p.at[:, 0].set(W_e[:D, 0]).at[:, 1].set(W_e[D:, 0])
    be_p = jnp.zeros((1, 8), jnp.float32).at[0, 0].set(b_e[0])

    ab = _tc3(h1, s_full[0], s_full[1], deg, W_self2, W_neigh2, b2,
              we_p, be_p)
    a = ab[:, 0]
    b = ab[:, 1]

    out = _edge_scores(src, dst, a, b)
    return out.reshape(E, 1)


# speedup vs baseline: 1.0016x; 1.0016x over previous
import functools

import jax
import jax.numpy as jnp
from jax import lax
from jax.experimental import pallas as pl
from jax.experimental.pallas import tpu as pltpu
from jax.experimental.pallas import tpu_sc as plsc

N = 10000
E = 320000
DIN = 512
D = 128

NC = 2
NS = 16
NW = NC * NS
RNG = 320
NPAD = NW * RNG
ACC_ROWS = 328
DUMP = 327
WAVE = 128
BCAP = 14720

E_PER_W = E // NW
SUM_WAVE = 80
SUM_NW = E_PER_W // SUM_WAVE
SPAD = 10240
SSTRIPE = SPAD // NS
SCHUNK = SSTRIPE // 2

BLK = 1000
GRID = N // BLK

_mesh = functools.partial(
    plsc.VectorSubcoreMesh, core_axis_name="c", subcore_axis_name="s")


def _lane():
    return lax.iota(jnp.int32, 16)


def _rup128(x):
    return (x + 127) & ~jnp.int32(127)


def _bin_body(src_hbm, dst_hbm, binned_out, counts_out, histn_out,
              sv_v, dv_v, bv_v, pk_v, hist, cur, buf, histn):
    c = lax.axis_index("c")
    s = lax.axis_index("s")
    wid = s * NC + c
    base_e = wid * E_PER_W

    pltpu.sync_copy(src_hbm.at[pl.ds(base_e, E_PER_W)],
                    sv_v.at[pl.ds(0, E_PER_W)])
    pltpu.sync_copy(dst_hbm.at[pl.ds(base_e, E_PER_W)],
                    dv_v.at[pl.ds(0, E_PER_W)])

    zi = jnp.zeros((16,), jnp.int32)
    one0 = jnp.where(_lane() == 0, 1, 0)
    for j in range(48 // 16):
        hist[pl.ds(j * 16, 16)] = zi

    def zh(i, _):
        histn[pl.ds(i * 16, 16)] = zi
        return 0
    lax.fori_loop(0, NPAD // 16, zh, 0)

    def prep(i, _):
        sl = pl.ds(i * 16, 16)
        d = dv_v[sl]
        b = lax.shift_right_logical(d * 6554, 21)
        bv_v[sl] = b
        pk_v[sl] = lax.shift_left(sv_v[sl], 9) | (d - b * RNG)
        return 0
    lax.fori_loop(0, E_PER_W // 16, prep, 0)

    def h1(e, _):
        d = dv_v[pl.ds(e, 16)][0]
        bsl = pl.ds(lax.shift_right_logical(d * 6554, 21), 16)
        hist[bsl] = hist[bsl] + one0
        dsl = pl.ds(d, 16)
        histn[dsl] = histn[dsl] + one0
        return 0
    lax.fori_loop(0, E_PER_W, h1, 0)

    def st(b, off):
        cur[pl.ds(b, 16)] = zi + off
        return off + _rup128(hist[pl.ds(b, 16)][0] + 16)
    lax.fori_loop(0, NW, st, 0)

    dumpv = jnp.full((16,), DUMP, jnp.int32)

    def zb(i, _):
        buf[pl.ds(i * 16, 16)] = dumpv
        return 0
    lax.fori_loop(0, BCAP // 16, zb, 0)

    lane0 = _lane() == 0

    def h2(e, _):
        b = bv_v[pl.ds(e, 16)][0]
        bsl = pl.ds(b, 16)
        cb = cur[bsl]
        o = cb[0]
        buf[pl.ds(o, 16)] = jnp.where(lane0, pk_v[pl.ds(e, 16)], dumpv)
        cur[bsl] = cb + one0
        return 0
    lax.fori_loop(0, E_PER_W, h2, 0)

    pltpu.sync_copy(buf.at[pl.ds(0, BCAP)],
                    binned_out.at[pl.ds(wid * BCAP, BCAP)])
    pltpu.sync_copy(hist.at[pl.ds(0, NW)],
                    counts_out.at[pl.ds(wid * NW, NW)])
    pltpu.sync_copy(histn.at[pl.ds(0, NPAD)],
                    histn_out.at[pl.ds(wid * NPAD, NPAD)])


def _bin_edges(src, dst):
    return pl.kernel(
        _bin_body,
        out_type=(
            jax.ShapeDtypeStruct((NW * BCAP,), jnp.int32),
            jax.ShapeDtypeStruct((NW * NW,), jnp.int32),
            jax.ShapeDtypeStruct((NW * NPAD,), jnp.int32),
        ),
        mesh=_mesh(),
        scratch_types=[
            pltpu.VMEM((E_PER_W + 16,), jnp.int32),
            pltpu.VMEM((E_PER_W + 16,), jnp.int32),
            pltpu.VMEM((E_PER_W + 16,), jnp.int32),
            pltpu.VMEM((E_PER_W + 16,), jnp.int32),
            pltpu.VMEM((48,), jnp.int32),
            pltpu.VMEM((48,), jnp.int32),
            pltpu.VMEM((BCAP + 16,), jnp.int32),
            pltpu.VMEM((NPAD + 16,), jnp.int32),
        ],
    )(src, dst)


WCAP = 2656


def _segmax_body(binned_hbm, counts_hbm, m_hbm, agg_out,
                 acc, cnt_v, woff,
                 pkb0, pkb1, widx0, widx1, rows0, rows1,
                 psem0, psem1, gsem0, gsem1):
    c = lax.axis_index("c")
    s = lax.axis_index("s")
    wid = s * NC + c

    zf = jnp.zeros((16,), jnp.float32)
    zi = jnp.zeros((16,), jnp.int32)

    def zero_row(r, _):
        for j in range(8):
            acc[r, pl.ds(j * 16, 16)] = zf
        return 0
    lax.fori_loop(0, ACC_ROWS, zero_row, 0)

    pltpu.sync_copy(counts_hbm, cnt_v.at[pl.ds(0, NW * NW)])

    def seg(t, nwt):
        def st(b, off):
            return off + _rup128(cnt_v[pl.ds(t * NW + b, 16)][0] + 16)
        offw = lax.fori_loop(0, wid, st, 0)
        cnt = cnt_v[pl.ds(t * NW + wid, 16)][0]
        nwv = (cnt + (WAVE - 1)) // WAVE

        def put(v, _):
            woff[pl.ds(nwt + v, 16)] = zi + (t * BCAP + offw + v * WAVE)
            return 0
        lax.fori_loop(0, nwv, put, 0)
        return nwt + nwv
    nw_tot = lax.fori_loop(0, NW, seg, 0)
    for k in range(3):
        woff[pl.ds(nw_tot + k * 16, 16)] = zi + (BCAP - WAVE)

    def fire_pk(w, dst, sem):
        off = pl.multiple_of(woff[pl.ds(w, 16)][0], WAVE)
        pltpu.async_copy(binned_hbm.at[pl.ds(off, WAVE)],
                         dst.at[pl.ds(0, WAVE)], sem)

    def wait_pk(dst, sem):
        pltpu.make_async_copy(binned_hbm.at[pl.ds(0, WAVE)],
                              dst.at[pl.ds(0, WAVE)], sem).wait()

    def unpack(pkb, widx):
        for j in range(WAVE // 16):
            sl = pl.ds(j * 16, 16)
            widx[sl] = lax.shift_right_logical(pkb[sl], 9)

    def accum(pkb, rows):
        def edge(e, _):
            dl = pkb[pl.ds(e, 16)][0] & 511
            for j in range(8):
                sl2 = pl.ds(j * 16, 16)
                acc[dl, sl2] = jnp.maximum(acc[dl, sl2], rows[e, sl2])
            return 0
        lax.fori_loop(0, WAVE, edge, 0)

    fire_pk(0, pkb0, psem0)
    npair = (nw_tot + 1) // 2

    def pair(p, _):
        w = p * 2
        wait_pk(pkb0, psem0)
        unpack(pkb0, widx0)
        pltpu.async_copy(m_hbm.at[widx0], rows0, gsem0)
        fire_pk(w + 1, pkb1, psem1)
        wait_pk(pkb1, psem1)
        unpack(pkb1, widx1)
        pltpu.make_async_copy(m_hbm.at[widx0], rows0, gsem0).wait()
        pltpu.async_copy(m_hbm.at[widx1], rows1, gsem1)
        accum(pkb0, rows0)
        pltpu.make_async_copy(m_hbm.at[widx1], rows1, gsem1).wait()
        accum(pkb1, rows1)
        fire_pk(w + 2, pkb0, psem0)
        return 0
    lax.fori_loop(0, npair, pair, 0)
    wait_pk(pkb0, psem0)

    pltpu.sync_copy(acc.at[pl.ds(0, RNG)], agg_out.at[wid])


def _segmax(binned, counts, m):
    return pl.kernel(
        _segmax_body,
        out_type=jax.ShapeDtypeStruct((NW, RNG, D), jnp.float32),
        mesh=_mesh(),
        scratch_types=[
            pltpu.VMEM((ACC_ROWS, D), jnp.float32),
            pltpu.VMEM((NW * NW + 16,), jnp.int32),
            pltpu.VMEM((WCAP + 64,), jnp.int32),
            pltpu.VMEM((WAVE,), jnp.int32),
            pltpu.VMEM((WAVE,), jnp.int32),
            pltpu.VMEM((WAVE,), jnp.int32),
            pltpu.VMEM((WAVE,), jnp.int32),
            pltpu.VMEM((WAVE, D), jnp.float32),
            pltpu.VMEM((WAVE, D), jnp.float32),
            pltpu.SemaphoreType.DMA,
            pltpu.SemaphoreType.DMA,
            pltpu.SemaphoreType.DMA,
            pltpu.SemaphoreType.DMA,
        ],
    )(binned, counts, m)


def _segsum_body(src_hbm, dst_hbm, h1_hbm, out_hbm,
                 shared, buf, sidx, didx, sem):
    c = lax.axis_index("c")
    s = lax.axis_index("s")
    wid = s * NC + c
    base_e = wid * E_PER_W

    zf = jnp.zeros((16,), jnp.float32)

    def zero_row(r, _):
        for j in range(8):
            buf[r, pl.ds(j * 16, 16)] = zf
        return 0
    lax.fori_loop(0, SCHUNK, zero_row, 0)
    for k in range(2):
        pltpu.sync_copy(buf,
                        shared.at[pl.ds(s * SSTRIPE + k * SCHUNK, SCHUNK)])
    plsc.subcore_barrier()

    def wave(w, _):
        e0 = base_e + w * SUM_WAVE
        pltpu.sync_copy(src_hbm.at[pl.ds(e0, SUM_WAVE)], sidx)
        pltpu.sync_copy(dst_hbm.at[pl.ds(e0, SUM_WAVE)], didx)
        pltpu.async_copy(h1_hbm.at[sidx], buf.at[pl.ds(0, SUM_WAVE)],
                         sem).wait()
        pltpu.sync_copy(buf.at[pl.ds(0, SUM_WAVE)], shared.at[didx],
                        add=True)
        return 0
    lax.fori_loop(0, SUM_NW, wave, 0)

    plsc.subcore_barrier()

    for k in range(2):
        r0 = s * SSTRIPE + k * SCHUNK
        pltpu.sync_copy(shared.at[pl.ds(r0, SCHUNK)], buf)
        pltpu.sync_copy(buf, out_hbm.at[c].at[pl.ds(r0, SCHUNK)])


def _segsum(src, dst, h1):
    return pl.kernel(
        _segsum_body,
        out_type=jax.ShapeDtypeStruct((NC, SPAD, D), jnp.float32),
        mesh=_mesh(),
        scratch_types=[
            pltpu.VMEM_SHARED((SPAD, D), jnp.float32),
            pltpu.VMEM((SCHUNK, D), jnp.float32),
            pltpu.VMEM((SUM_WAVE,), jnp.int32),
            pltpu.VMEM((SUM_WAVE,), jnp.int32),
            pltpu.SemaphoreType.DMA,
        ],
    )(src, dst, h1)


def _edge_body(src_hbm, dst_hbm, a_hbm, b_hbm, out_hbm,
               a_v, b_v, s_v, d_v, o_v):
    c = lax.axis_index("c")
    s = lax.axis_index("s")
    wid = s * NC + c
    base_e = wid * E_PER_W

    pltpu.sync_copy(a_hbm, a_v.at[pl.ds(0, N)])
    pltpu.sync_copy(b_hbm, b_v.at[pl.ds(0, N)])
    pltpu.sync_copy(src_hbm.at[pl.ds(base_e, E_PER_W)],
                    s_v.at[pl.ds(0, E_PER_W)])
    pltpu.sync_copy(dst_hbm.at[pl.ds(base_e, E_PER_W)],
                    d_v.at[pl.ds(0, E_PER_W)])

    def lp(e, _):
        si = s_v[pl.ds(e, 16)][0]
        di = d_v[pl.ds(e, 16)][0]
        va = a_v[pl.ds(si, 16)][0]
        vb = b_v[pl.ds(di, 16)][0]
        o_v[pl.ds(e, 16)] = jnp.zeros((16,), jnp.float32) + (va + vb)
        return 0
    lax.fori_loop(0, E_PER_W, lp, 0)

    pltpu.sync_copy(o_v.at[pl.ds(0, E_PER_W)],
                    out_hbm.at[pl.ds(base_e, E_PER_W)])


def _edge_scores(src, dst, a, b):
    return pl.kernel(
        _edge_body,
        out_type=jax.ShapeDtypeStruct((E,), jnp.float32),
        mesh=_mesh(),
        scratch_types=[
            pltpu.VMEM((N + 16,), jnp.float32),
            pltpu.VMEM((N + 16,), jnp.float32),
            pltpu.VMEM((E_PER_W + 16,), jnp.int32),
            pltpu.VMEM((E_PER_W + 16,), jnp.int32),
            pltpu.VMEM((E_PER_W + 16,), jnp.float32),
        ],
    )(src, dst, a, b)


def _tc1_body(x_ref, wr_ref, br_ref, wp_ref, bp_ref, h_ref, m_ref):
    h = jnp.dot(x_ref[...], wr_ref[...],
                preferred_element_type=jnp.float32) + br_ref[...]
    h_ref[...] = h
    m = jnp.dot(h, wp_ref[...], preferred_element_type=jnp.float32)
    m_ref[...] = jnp.maximum(m + bp_ref[...], 0.0)


def _tc1(x, W_red, b_red, W_pool, b_pool):
    return pl.pallas_call(
        _tc1_body,
        grid=(GRID,),
        in_specs=[
            pl.BlockSpec((BLK, DIN), lambda i: (i, 0)),
            pl.BlockSpec((DIN, D), lambda i: (0, 0)),
            pl.BlockSpec((1, D), lambda i: (0, 0)),
            pl.BlockSpec((D, D), lambda i: (0, 0)),
            pl.BlockSpec((1, D), lambda i: (0, 0)),
        ],
        out_specs=[
            pl.BlockSpec((BLK, D), lambda i: (i, 0)),
            pl.BlockSpec((BLK, D), lambda i: (i, 0)),
        ],
        out_shape=[
            jax.ShapeDtypeStruct((N, D), jnp.float32),
            jax.ShapeDtypeStruct((N, D), jnp.float32),
        ],
    )(x, W_red, b_red.reshape(1, D), W_pool, b_pool.reshape(1, D))


def _tc2_body(h_ref, agg_ref, ws_ref, wn_ref, b_ref, hist_ref,
              h1_ref, deg_ref):
    r = (jnp.dot(h_ref[...], ws_ref[...], preferred_element_type=jnp.float32)
         + jnp.dot(agg_ref[...], wn_ref[...],
                   preferred_element_type=jnp.float32)
         + b_ref[...])
    n = jnp.sqrt(jnp.sum(r * r, axis=-1, keepdims=True))
    r = r / jnp.maximum(n, 1e-12)
    h1_ref[...] = jnp.maximum(r, 0.0)
    deg_ref[...] = jnp.sum(hist_ref[...].astype(jnp.float32), axis=1,
                           keepdims=True)


def _tc2(h, agg, W_self1, W_neigh1, b1, hists):
    return pl.pallas_call(
        _tc2_body,
        grid=(GRID,),
        in_specs=[
            pl.BlockSpec((BLK, D), lambda i: (i, 0)),
            pl.BlockSpec((BLK, D), lambda i: (i, 0)),
            pl.BlockSpec((D, D), lambda i: (0, 0)),
            pl.BlockSpec((D, D), lambda i: (0, 0)),
            pl.BlockSpec((1, D), lambda i: (0, 0)),
            pl.BlockSpec((BLK, NW), lambda i: (i, 0)),
        ],
        out_specs=[
            pl.BlockSpec((BLK, D), lambda i: (i, 0)),
            pl.BlockSpec((BLK, 1), lambda i: (i, 0)),
        ],
        out_shape=[
            jax.ShapeDtypeStruct((N, D), jnp.float32),
            jax.ShapeDtypeStruct((N, 1), jnp.float32),
        ],
    )(h, agg, W_self1, W_neigh1, b1.reshape(1, D), hists)


def _tc3_body(h1_ref, s0_ref, s1_ref, deg_ref, ws_ref, wn_ref, b_ref,
              we_ref, be_ref, ab_ref):
    mean = (s0_ref[...] + s1_ref[...]) / jnp.maximum(deg_ref[...], 1.0)
    h2 = (jnp.dot(h1_ref[...], ws_ref[...],
                  preferred_element_type=jnp.float32)
          + jnp.dot(mean, wn_ref[...], preferred_element_type=jnp.float32)
          + b_ref[...])
    h2 = jnp.maximum(h2, 0.0)
    ab = jnp.dot(h2, we_ref[...], preferred_element_type=jnp.float32)
    ab_ref[...] = ab + be_ref[...]


def _tc3(h1, s0, s1, deg, W_self2, W_neigh2, b2, we_p, be_p):
    return pl.pallas_call(
        _tc3_body,
        grid=(GRID,),
        in_specs=[
            pl.BlockSpec((BLK, D), lambda i: (i, 0)),
            pl.BlockSpec((BLK, D), lambda i: (i, 0)),
            pl.BlockSpec((BLK, D), lambda i: (i, 0)),
            pl.BlockSpec((BLK, 1), lambda i: (i, 0)),
            pl.BlockSpec((D, D), lambda i: (0, 0)),
            pl.BlockSpec((D, D), lambda i: (0, 0)),
            pl.BlockSpec((1, D), lambda i: (0, 0)),
            pl.BlockSpec((D, 8), lambda i: (0, 0)),
            pl.BlockSpec((1, 8), lambda i: (0, 0)),
        ],
        out_specs=pl.BlockSpec((BLK, 8), lambda i: (i, 0)),
        out_shape=jax.ShapeDtypeStruct((N, 8), jnp.float32),
    )(h1, s0, s1, deg, W_self2, W_neigh2, b2.reshape(1, D), we_p, be_p)


def kernel(x, edge_index, W_red, b_red, W_pool, b_pool, W_self1, W_neigh1,
           b1, W_self2, W_neigh2, b2, W_e, b_e):
    src = edge_index[0]
    dst = edge_index[1]

    h, m = _tc1(x, W_red, b_red, W_pool, b_pool)

    binned, counts, histn = _bin_edges(src, dst)
    agg_t = _segmax(binned, counts, m)
    agg = agg_t.reshape(NPAD, D)[:N]
    hists = histn.reshape(NW, NPAD)[:, :N].T

    h1, deg = _tc2(h, agg, W_self1, W_neigh1, b1, hists)

    s_part = _segsum(src, dst, h1)
    s_full = s_part[:, :N, :]

    we_p = jnp.zeros((D, 8), jnp.float32)
    we_p = we_p.at[:, 0].set(W_e[:D, 0]).at[:, 1].set(W_e[D:, 0])
    be_p = jnp.zeros((1, 8), jnp.float32).at[0, 0].set(b_e[0])

    ab = _tc3(h1, s_full[0], s_full[1], deg, W_self2, W_neigh2, b2,
              we_p, be_p)
    a = ab[:, 0]
    b = ab[:, 1]

    out = _edge_scores(src, dst, a, b)
    return out.reshape(E, 1)


# speedup vs baseline: 1.0777x; 1.0760x over previous
import functools

import jax
import jax.numpy as jnp
from jax import lax
from jax.experimental import pallas as pl
from jax.experimental.pallas import tpu as pltpu
from jax.experimental.pallas import tpu_sc as plsc

N = 10000
E = 320000
DIN = 512
D = 128

NC = 2
NS = 16
NW = NC * NS
RNG = 320
NPAD = NW * RNG
ACC_ROWS = 328
DUMP = 327
WAVE = 128
BCAP = 14720

E_PER_W = E // NW
SUM_WAVE = 80
SUM_NW = E_PER_W // SUM_WAVE
SPAD = 10240
SSTRIPE = SPAD // NS
SCHUNK = SSTRIPE // 2

BLK = 1000
GRID = N // BLK

_mesh = functools.partial(
    plsc.VectorSubcoreMesh, core_axis_name="c", subcore_axis_name="s")


def _lane():
    return lax.iota(jnp.int32, 16)


def _rup128(x):
    return (x + 127) & ~jnp.int32(127)


def _bin_body(src_hbm, dst_hbm, binned_out, counts_out, histn_out,
              sv_v, dv_v, bv_v, pk_v, hist, cur, buf, histn):
    c = lax.axis_index("c")
    s = lax.axis_index("s")
    wid = s * NC + c
    base_e = wid * E_PER_W

    pltpu.sync_copy(src_hbm.at[pl.ds(base_e, E_PER_W)],
                    sv_v.at[pl.ds(0, E_PER_W)])
    pltpu.sync_copy(dst_hbm.at[pl.ds(base_e, E_PER_W)],
                    dv_v.at[pl.ds(0, E_PER_W)])

    zi = jnp.zeros((16,), jnp.int32)
    one0 = jnp.where(_lane() == 0, 1, 0)
    for j in range(48 // 16):
        hist[pl.ds(j * 16, 16)] = zi

    def zh(i, _):
        histn[pl.ds(i * 16, 16)] = zi
        return 0
    lax.fori_loop(0, NPAD // 16, zh, 0)

    def prep(i, _):
        sl = pl.ds(i * 16, 16)
        d = dv_v[sl]
        b = lax.shift_right_logical(d * 6554, 21)
        bv_v[sl] = b
        pk_v[sl] = lax.shift_left(sv_v[sl], 9) | (d - b * RNG)
        return 0
    lax.fori_loop(0, E_PER_W // 16, prep, 0)

    def h1(g, _):
        d_vec = dv_v[pl.ds(g * 16, 16)]
        b_vec = lax.shift_right_logical(d_vec * 6554, 21)
        for k in range(16):
            bsl = pl.ds(b_vec[k], 16)
            hist[bsl] = hist[bsl] + one0
            dsl = pl.ds(d_vec[k], 16)
            histn[dsl] = histn[dsl] + one0
        return 0
    lax.fori_loop(0, E_PER_W // 16, h1, 0)

    def st(b, off):
        cur[pl.ds(b, 16)] = zi + off
        return off + _rup128(hist[pl.ds(b, 16)][0] + 16)
    lax.fori_loop(0, NW, st, 0)

    dumpv = jnp.full((16,), DUMP, jnp.int32)

    def zb(i, _):
        buf[pl.ds(i * 16, 16)] = dumpv
        return 0
    lax.fori_loop(0, BCAP // 16, zb, 0)

    lane0 = _lane() == 0

    def h2(g, _):
        b_vec = bv_v[pl.ds(g * 16, 16)]
        pv = pk_v[pl.ds(g * 16, 16)]
        for k in range(16):
            bsl = pl.ds(b_vec[k], 16)
            cb = cur[bsl]
            o = cb[0]
            buf[pl.ds(o, 16)] = jnp.where(lane0, pv[k], dumpv)
            cur[bsl] = cb + one0
        return 0
    lax.fori_loop(0, E_PER_W // 16, h2, 0)

    pltpu.sync_copy(buf.at[pl.ds(0, BCAP)],
                    binned_out.at[pl.ds(wid * BCAP, BCAP)])
    pltpu.sync_copy(hist.at[pl.ds(0, NW)],
                    counts_out.at[pl.ds(wid * NW, NW)])
    pltpu.sync_copy(histn.at[pl.ds(0, NPAD)],
                    histn_out.at[pl.ds(wid * NPAD, NPAD)])


def _bin_edges(src, dst):
    return pl.kernel(
        _bin_body,
        out_type=(
            jax.ShapeDtypeStruct((NW * BCAP,), jnp.int32),
            jax.ShapeDtypeStruct((NW * NW,), jnp.int32),
            jax.ShapeDtypeStruct((NW * NPAD,), jnp.int32),
        ),
        mesh=_mesh(),
        scratch_types=[
            pltpu.VMEM((E_PER_W + 16,), jnp.int32),
            pltpu.VMEM((E_PER_W + 16,), jnp.int32),
            pltpu.VMEM((E_PER_W + 16,), jnp.int32),
            pltpu.VMEM((E_PER_W + 16,), jnp.int32),
            pltpu.VMEM((48,), jnp.int32),
            pltpu.VMEM((48,), jnp.int32),
            pltpu.VMEM((BCAP + 16,), jnp.int32),
            pltpu.VMEM((NPAD + 16,), jnp.int32),
        ],
    )(src, dst)


WCAP = 2656


def _segmax_body(binned_hbm, counts_hbm, m_hbm, agg_out,
                 acc, cnt_v, woff,
                 pkb0, pkb1, widx0, widx1, rows0, rows1,
                 psem0, psem1, gsem0, gsem1):
    c = lax.axis_index("c")
    s = lax.axis_index("s")
    wid = s * NC + c

    zf = jnp.zeros((16,), jnp.float32)
    zi = jnp.zeros((16,), jnp.int32)

    def zero_row(r, _):
        for j in range(8):
            acc[r, pl.ds(j * 16, 16)] = zf
        return 0
    lax.fori_loop(0, ACC_ROWS, zero_row, 0)

    pltpu.sync_copy(counts_hbm, cnt_v.at[pl.ds(0, NW * NW)])

    def seg(t, nwt):
        def st(b, off):
            return off + _rup128(cnt_v[pl.ds(t * NW + b, 16)][0] + 16)
        offw = lax.fori_loop(0, wid, st, 0)
        cnt = cnt_v[pl.ds(t * NW + wid, 16)][0]
        nwv = (cnt + (WAVE - 1)) // WAVE

        def put(v, _):
            woff[pl.ds(nwt + v, 16)] = zi + (t * BCAP + offw + v * WAVE)
            return 0
        lax.fori_loop(0, nwv, put, 0)
        return nwt + nwv
    nw_tot = lax.fori_loop(0, NW, seg, 0)
    for k in range(3):
        woff[pl.ds(nw_tot + k * 16, 16)] = zi + (BCAP - WAVE)

    def fire_pk(w, dst, sem):
        off = pl.multiple_of(woff[pl.ds(w, 16)][0], WAVE)
        pltpu.async_copy(binned_hbm.at[pl.ds(off, WAVE)],
                         dst.at[pl.ds(0, WAVE)], sem)

    def wait_pk(dst, sem):
        pltpu.make_async_copy(binned_hbm.at[pl.ds(0, WAVE)],
                              dst.at[pl.ds(0, WAVE)], sem).wait()

    def unpack(pkb, widx):
        for j in range(WAVE // 16):
            sl = pl.ds(j * 16, 16)
            widx[sl] = lax.shift_right_logical(pkb[sl], 9)

    def accum(pkb, rows):
        def grp(g, _):
            dl_vec = pkb[pl.ds(g * 16, 16)] & 511
            for k in range(16):
                d = dl_vec[k]
                e = g * 16 + k
                for j in range(8):
                    sl2 = pl.ds(j * 16, 16)
                    acc[d, sl2] = jnp.maximum(acc[d, sl2], rows[e, sl2])
            return 0
        lax.fori_loop(0, WAVE // 16, grp, 0)

    fire_pk(0, pkb0, psem0)
    npair = (nw_tot + 1) // 2

    def pair(p, _):
        w = p * 2
        wait_pk(pkb0, psem0)
        unpack(pkb0, widx0)
        pltpu.async_copy(m_hbm.at[widx0], rows0, gsem0)
        fire_pk(w + 1, pkb1, psem1)
        wait_pk(pkb1, psem1)
        unpack(pkb1, widx1)
        pltpu.make_async_copy(m_hbm.at[widx0], rows0, gsem0).wait()
        pltpu.async_copy(m_hbm.at[widx1], rows1, gsem1)
        accum(pkb0, rows0)
        pltpu.make_async_copy(m_hbm.at[widx1], rows1, gsem1).wait()
        accum(pkb1, rows1)
        fire_pk(w + 2, pkb0, psem0)
        return 0
    lax.fori_loop(0, npair, pair, 0)
    wait_pk(pkb0, psem0)

    pltpu.sync_copy(acc.at[pl.ds(0, RNG)], agg_out.at[wid])


def _segmax(binned, counts, m):
    return pl.kernel(
        _segmax_body,
        out_type=jax.ShapeDtypeStruct((NW, RNG, D), jnp.float32),
        mesh=_mesh(),
        scratch_types=[
            pltpu.VMEM((ACC_ROWS, D), jnp.float32),
            pltpu.VMEM((NW * NW + 16,), jnp.int32),
            pltpu.VMEM((WCAP + 64,), jnp.int32),
            pltpu.VMEM((WAVE,), jnp.int32),
            pltpu.VMEM((WAVE,), jnp.int32),
            pltpu.VMEM((WAVE,), jnp.int32),
            pltpu.VMEM((WAVE,), jnp.int32),
            pltpu.VMEM((WAVE, D), jnp.float32),
            pltpu.VMEM((WAVE, D), jnp.float32),
            pltpu.SemaphoreType.DMA,
            pltpu.SemaphoreType.DMA,
            pltpu.SemaphoreType.DMA,
            pltpu.SemaphoreType.DMA,
        ],
    )(binned, counts, m)


def _segsum_body(src_hbm, dst_hbm, h1_hbm, out_hbm,
                 shared, buf, sidx, didx, sem):
    c = lax.axis_index("c")
    s = lax.axis_index("s")
    wid = s * NC + c
    base_e = wid * E_PER_W

    zf = jnp.zeros((16,), jnp.float32)

    def zero_row(r, _):
        for j in range(8):
            buf[r, pl.ds(j * 16, 16)] = zf
        return 0
    lax.fori_loop(0, SCHUNK, zero_row, 0)
    for k in range(2):
        pltpu.sync_copy(buf,
                        shared.at[pl.ds(s * SSTRIPE + k * SCHUNK, SCHUNK)])
    plsc.subcore_barrier()

    def wave(w, _):
        e0 = base_e + w * SUM_WAVE
        pltpu.sync_copy(src_hbm.at[pl.ds(e0, SUM_WAVE)], sidx)
        pltpu.sync_copy(dst_hbm.at[pl.ds(e0, SUM_WAVE)], didx)
        pltpu.async_copy(h1_hbm.at[sidx], buf.at[pl.ds(0, SUM_WAVE)],
                         sem).wait()
        pltpu.sync_copy(buf.at[pl.ds(0, SUM_WAVE)], shared.at[didx],
                        add=True)
        return 0
    lax.fori_loop(0, SUM_NW, wave, 0)

    plsc.subcore_barrier()

    for k in range(2):
        r0 = s * SSTRIPE + k * SCHUNK
        pltpu.sync_copy(shared.at[pl.ds(r0, SCHUNK)], buf)
        pltpu.sync_copy(buf, out_hbm.at[c].at[pl.ds(r0, SCHUNK)])


def _segsum(src, dst, h1):
    return pl.kernel(
        _segsum_body,
        out_type=jax.ShapeDtypeStruct((NC, SPAD, D), jnp.float32),
        mesh=_mesh(),
        scratch_types=[
            pltpu.VMEM_SHARED((SPAD, D), jnp.float32),
            pltpu.VMEM((SCHUNK, D), jnp.float32),
            pltpu.VMEM((SUM_WAVE,), jnp.int32),
            pltpu.VMEM((SUM_WAVE,), jnp.int32),
            pltpu.SemaphoreType.DMA,
        ],
    )(src, dst, h1)


def _edge_body(src_hbm, dst_hbm, a_hbm, b_hbm, out_hbm,
               a_v, b_v, s_v, d_v, o_v):
    c = lax.axis_index("c")
    s = lax.axis_index("s")
    wid = s * NC + c
    base_e = wid * E_PER_W

    pltpu.sync_copy(a_hbm, a_v.at[pl.ds(0, N)])
    pltpu.sync_copy(b_hbm, b_v.at[pl.ds(0, N)])
    pltpu.sync_copy(src_hbm.at[pl.ds(base_e, E_PER_W)],
                    s_v.at[pl.ds(0, E_PER_W)])
    pltpu.sync_copy(dst_hbm.at[pl.ds(base_e, E_PER_W)],
                    d_v.at[pl.ds(0, E_PER_W)])

    def lp(e, _):
        si = s_v[pl.ds(e, 16)][0]
        di = d_v[pl.ds(e, 16)][0]
        va = a_v[pl.ds(si, 16)][0]
        vb = b_v[pl.ds(di, 16)][0]
        o_v[pl.ds(e, 16)] = jnp.zeros((16,), jnp.float32) + (va + vb)
        return 0
    lax.fori_loop(0, E_PER_W, lp, 0)

    pltpu.sync_copy(o_v.at[pl.ds(0, E_PER_W)],
                    out_hbm.at[pl.ds(base_e, E_PER_W)])


def _edge_scores(src, dst, a, b):
    return pl.kernel(
        _edge_body,
        out_type=jax.ShapeDtypeStruct((E,), jnp.float32),
        mesh=_mesh(),
        scratch_types=[
            pltpu.VMEM((N + 16,), jnp.float32),
            pltpu.VMEM((N + 16,), jnp.float32),
            pltpu.VMEM((E_PER_W + 16,), jnp.int32),
            pltpu.VMEM((E_PER_W + 16,), jnp.int32),
            pltpu.VMEM((E_PER_W + 16,), jnp.float32),
        ],
    )(src, dst, a, b)


def _tc1_body(x_ref, wr_ref, br_ref, wp_ref, bp_ref, h_ref, m_ref):
    h = jnp.dot(x_ref[...], wr_ref[...],
                preferred_element_type=jnp.float32) + br_ref[...]
    h_ref[...] = h
    m = jnp.dot(h, wp_ref[...], preferred_element_type=jnp.float32)
    m_ref[...] = jnp.maximum(m + bp_ref[...], 0.0)


def _tc1(x, W_red, b_red, W_pool, b_pool):
    return pl.pallas_call(
        _tc1_body,
        grid=(GRID,),
        in_specs=[
            pl.BlockSpec((BLK, DIN), lambda i: (i, 0)),
            pl.BlockSpec((DIN, D), lambda i: (0, 0)),
            pl.BlockSpec((1, D), lambda i: (0, 0)),
            pl.BlockSpec((D, D), lambda i: (0, 0)),
            pl.BlockSpec((1, D), lambda i: (0, 0)),
        ],
        out_specs=[
            pl.BlockSpec((BLK, D), lambda i: (i, 0)),
            pl.BlockSpec((BLK, D), lambda i: (i, 0)),
        ],
        out_shape=[
            jax.ShapeDtypeStruct((N, D), jnp.float32),
            jax.ShapeDtypeStruct((N, D), jnp.float32),
        ],
    )(x, W_red, b_red.reshape(1, D), W_pool, b_pool.reshape(1, D))


def _tc2_body(h_ref, agg_ref, ws_ref, wn_ref, b_ref, hist_ref,
              h1_ref, deg_ref):
    r = (jnp.dot(h_ref[...], ws_ref[...], preferred_element_type=jnp.float32)
         + jnp.dot(agg_ref[...], wn_ref[...],
                   preferred_element_type=jnp.float32)
         + b_ref[...])
    n = jnp.sqrt(jnp.sum(r * r, axis=-1, keepdims=True))
    r = r / jnp.maximum(n, 1e-12)
    h1_ref[...] = jnp.maximum(r, 0.0)
    deg_ref[...] = jnp.sum(hist_ref[...].astype(jnp.float32), axis=1,
                           keepdims=True)


def _tc2(h, agg, W_self1, W_neigh1, b1, hists):
    return pl.pallas_call(
        _tc2_body,
        grid=(GRID,),
        in_specs=[
            pl.BlockSpec((BLK, D), lambda i: (i, 0)),
            pl.BlockSpec((BLK, D), lambda i: (i, 0)),
            pl.BlockSpec((D, D), lambda i: (0, 0)),
            pl.BlockSpec((D, D), lambda i: (0, 0)),
            pl.BlockSpec((1, D), lambda i: (0, 0)),
            pl.BlockSpec((BLK, NW), lambda i: (i, 0)),
        ],
        out_specs=[
            pl.BlockSpec((BLK, D), lambda i: (i, 0)),
            pl.BlockSpec((BLK, 1), lambda i: (i, 0)),
        ],
        out_shape=[
            jax.ShapeDtypeStruct((N, D), jnp.float32),
            jax.ShapeDtypeStruct((N, 1), jnp.float32),
        ],
    )(h, agg, W_self1, W_neigh1, b1.reshape(1, D), hists)


def _tc3_body(h1_ref, s0_ref, s1_ref, deg_ref, ws_ref, wn_ref, b_ref,
              we_ref, be_ref, ab_ref):
    mean = (s0_ref[...] + s1_ref[...]) / jnp.maximum(deg_ref[...], 1.0)
    h2 = (jnp.dot(h1_ref[...], ws_ref[...],
                  preferred_element_type=jnp.float32)
          + jnp.dot(mean, wn_ref[...], preferred_element_type=jnp.float32)
          + b_ref[...])
    h2 = jnp.maximum(h2, 0.0)
    ab = jnp.dot(h2, we_ref[...], preferred_element_type=jnp.float32)
    ab_ref[...] = ab + be_ref[...]


def _tc3(h1, s0, s1, deg, W_self2, W_neigh2, b2, we_p, be_p):
    return pl.pallas_call(
        _tc3_body,
        grid=(GRID,),
        in_specs=[
            pl.BlockSpec((BLK, D), lambda i: (i, 0)),
            pl.BlockSpec((BLK, D), lambda i: (i, 0)),
            pl.BlockSpec((BLK, D), lambda i: (i, 0)),
            pl.BlockSpec((BLK, 1), lambda i: (i, 0)),
            pl.BlockSpec((D, D), lambda i: (0, 0)),
            pl.BlockSpec((D, D), lambda i: (0, 0)),
            pl.BlockSpec((1, D), lambda i: (0, 0)),
            pl.BlockSpec((D, 8), lambda i: (0, 0)),
            pl.BlockSpec((1, 8), lambda i: (0, 0)),
        ],
        out_specs=pl.BlockSpec((BLK, 8), lambda i: (i, 0)),
        out_shape=jax.ShapeDtypeStruct((N, 8), jnp.float32),
    )(h1, s0, s1, deg, W_self2, W_neigh2, b2.reshape(1, D), we_p, be_p)


def kernel(x, edge_index, W_red, b_red, W_pool, b_pool, W_self1, W_neigh1,
           b1, W_self2, W_neigh2, b2, W_e, b_e):
    src = edge_index[0]
    dst = edge_index[1]

    h, m = _tc1(x, W_red, b_red, W_pool, b_pool)

    binned, counts, histn = _bin_edges(src, dst)
    agg_t = _segmax(binned, counts, m)
    agg = agg_t.reshape(NPAD, D)[:N]
    hists = histn.reshape(NW, NPAD)[:, :N].T

    h1, deg = _tc2(h, agg, W_self1, W_neigh1, b1, hists)

    s_part = _segsum(src, dst, h1)
    s_full = s_part[:, :N, :]

    we_p = jnp.zeros((D, 8), jnp.float32)
    we_p = we_p.at[:, 0].set(W_e[:D, 0]).at[:, 1].set(W_e[D:, 0])
    be_p = jnp.zeros((1, 8), jnp.float32).at[0, 0].set(b_e[0])

    ab = _tc3(h1, s_full[0], s_full[1], deg, W_self2, W_neigh2, b2,
              we_p, be_p)
    a = ab[:, 0]
    b = ab[:, 1]

    out = _edge_scores(src, dst, a, b)
    return out.reshape(E, 1)


# speedup vs baseline: 1.9326x; 1.7934x over previous
import functools

import jax
import jax.numpy as jnp
from jax import lax
from jax.experimental import pallas as pl
from jax.experimental.pallas import tpu as pltpu
from jax.experimental.pallas import tpu_sc as plsc

N = 10000
E = 320000
DIN = 512
D = 128

NC = 2
NS = 16
NW = NC * NS
RNG = 313
NPAD = NW * RNG
ACC_ROWS = 320
DUMP = 319
CHUNK = 4000
NCHUNK = E // CHUNK
WAVE = 128
KEEP_CAP = CHUNK + 2 * WAVE

E_PER_W = E // NW
SUM_WAVE = 80
SUM_NW = E_PER_W // SUM_WAVE
SPAD = 10240
SSTRIPE = SPAD // NS
SCHUNK = SSTRIPE // 2

BLK = 1000
GRID = N // BLK

_mesh = functools.partial(
    plsc.VectorSubcoreMesh, core_axis_name="c", subcore_axis_name="s")


def _lane():
    return lax.iota(jnp.int32, 16)


def _compact(msk, vals):
    lane = _lane()
    p = jnp.where(msk, 1, 0)
    for sh in (1, 2, 4, 8):
        g = p.at[jnp.where(lane >= sh, lane - sh, 0)].get(
            mode="promise_in_bounds")
        p = p + jnp.where(lane >= sh, g, 0)
    cnt = p[15]
    j = jnp.zeros((16,), jnp.int32)
    for sh in (8, 4, 2, 1):
        t = p.at[j + (sh - 1)].get(mode="promise_in_bounds")
        j = j + jnp.where(t < lane + 1, sh, 0)
    return cnt, [v.at[j].get(mode="promise_in_bounds") for v in vals]


def _segmax_body(src_hbm, dst_hbm, m_hbm, agg_out, deg_out,
                 acc, deg, dst_v, src_v, keep_s, keep_d, widx, wdl,
                 rows, sem):
    c = lax.axis_index("c")
    s = lax.axis_index("s")
    wid = s * NC + c
    lo = wid * RNG
    hi = lo + RNG

    zf = jnp.zeros((16,), jnp.float32)

    def zero_row(r, _):
        for j in range(8):
            acc[r, pl.ds(j * 16, 16)] = zf
        return 0
    lax.fori_loop(0, ACC_ROWS, zero_row, 0)
    for j in range((ACC_ROWS + 16) // 16):
        deg[pl.ds(j * 16, 16)] = zf

    one0 = jnp.where(_lane() == 0, 1.0, 0.0)

    def wave(base):
        for j in range(WAVE // 16):
            widx[pl.ds(j * 16, 16)] = keep_s[pl.ds(base + j * 16, 16)]
            wdl[pl.ds(j * 16, 16)] = keep_d[pl.ds(base + j * 16, 16)]
        pltpu.async_copy(m_hbm.at[widx], rows, sem).wait()

        def grp(g, _):
            dl_vec = wdl[pl.ds(g * 16, 16)]
            for k in range(16):
                dl = dl_vec[k]
                e = g * 16 + k
                dsl = pl.ds(dl, 16)
                deg[dsl] = deg[dsl] + one0
                for j in range(8):
                    sl = pl.ds(j * 16, 16)
                    acc[dl, sl] = jnp.maximum(acc[dl, sl], rows[e, sl])
            return 0
        lax.fori_loop(0, WAVE // 16, grp, 0)

    def chunk_body(ch, kept):
        pltpu.sync_copy(dst_hbm.at[pl.ds(ch * CHUNK, CHUNK)], dst_v)
        pltpu.sync_copy(src_hbm.at[pl.ds(ch * CHUNK, CHUNK)], src_v)

        def filt(i, kept):
            sl = pl.ds(i * 16, 16)
            d = dst_v[sl]
            sv = src_v[sl]
            msk = (d >= lo) & (d < hi)
            cnt, comp = _compact(msk, [sv, d - lo])
            keep_s[pl.ds(kept, 16)] = comp[0]
            keep_d[pl.ds(kept, 16)] = comp[1]
            return kept + cnt
        kept = lax.fori_loop(0, CHUNK // 16, filt, kept)

        nw = kept // WAVE

        def do_wave(w, _):
            wave(w * WAVE)
            return 0
        lax.fori_loop(0, nw, do_wave, 0)

        base = nw * WAVE

        @pl.when(nw > 0)
        def _move():
            for j in range(WAVE // 16):
                keep_s[pl.ds(j * 16, 16)] = keep_s[pl.ds(base + j * 16, 16)]
                keep_d[pl.ds(j * 16, 16)] = keep_d[pl.ds(base + j * 16, 16)]

        return kept - base

    kept = lax.fori_loop(0, NCHUNK, chunk_body, 0)

    zi = jnp.zeros((16,), jnp.int32)
    df = jnp.full((16,), DUMP, jnp.int32)
    for j in range(WAVE // 16):
        keep_s[pl.ds(kept + j * 16, 16)] = zi
        keep_d[pl.ds(kept + j * 16, 16)] = df
    wave(0)

    pltpu.sync_copy(acc.at[pl.ds(0, RNG)], agg_out.at[wid])
    pltpu.sync_copy(deg.at[pl.ds(0, ACC_ROWS)],
                    deg_out.at[pl.ds(wid * ACC_ROWS, ACC_ROWS)])


def _segmax(src, dst, m):
    return pl.kernel(
        _segmax_body,
        out_type=(
            jax.ShapeDtypeStruct((NW, RNG, D), jnp.float32),
            jax.ShapeDtypeStruct((NW * ACC_ROWS,), jnp.float32),
        ),
        mesh=_mesh(),
        scratch_types=[
            pltpu.VMEM((ACC_ROWS, D), jnp.float32),
            pltpu.VMEM((ACC_ROWS + 16,), jnp.float32),
            pltpu.VMEM((CHUNK,), jnp.int32),
            pltpu.VMEM((CHUNK,), jnp.int32),
            pltpu.VMEM((KEEP_CAP,), jnp.int32),
            pltpu.VMEM((KEEP_CAP,), jnp.int32),
            pltpu.VMEM((WAVE,), jnp.int32),
            pltpu.VMEM((WAVE + 16,), jnp.int32),
            pltpu.VMEM((WAVE, D), jnp.float32),
            pltpu.SemaphoreType.DMA,
        ],
    )(src, dst, m)


def _segsum_body(src_hbm, dst_hbm, h1_hbm, out_hbm,
                 shared, buf, sidx, didx, sem):
    c = lax.axis_index("c")
    s = lax.axis_index("s")
    wid = s * NC + c
    base_e = wid * E_PER_W

    zf = jnp.zeros((16,), jnp.float32)

    def zero_row(r, _):
        for j in range(8):
            buf[r, pl.ds(j * 16, 16)] = zf
        return 0
    lax.fori_loop(0, SCHUNK, zero_row, 0)
    for k in range(2):
        pltpu.sync_copy(buf,
                        shared.at[pl.ds(s * SSTRIPE + k * SCHUNK, SCHUNK)])
    plsc.subcore_barrier()

    def wave(w, _):
        e0 = base_e + w * SUM_WAVE
        pltpu.sync_copy(src_hbm.at[pl.ds(e0, SUM_WAVE)], sidx)
        pltpu.sync_copy(dst_hbm.at[pl.ds(e0, SUM_WAVE)], didx)
        pltpu.async_copy(h1_hbm.at[sidx], buf.at[pl.ds(0, SUM_WAVE)],
                         sem).wait()
        pltpu.sync_copy(buf.at[pl.ds(0, SUM_WAVE)], shared.at[didx],
                        add=True)
        return 0
    lax.fori_loop(0, SUM_NW, wave, 0)

    plsc.subcore_barrier()

    for k in range(2):
        r0 = s * SSTRIPE + k * SCHUNK
        pltpu.sync_copy(shared.at[pl.ds(r0, SCHUNK)], buf)
        pltpu.sync_copy(buf, out_hbm.at[c].at[pl.ds(r0, SCHUNK)])


def _segsum(src, dst, h1):
    return pl.kernel(
        _segsum_body,
        out_type=jax.ShapeDtypeStruct((NC, SPAD, D), jnp.float32),
        mesh=_mesh(),
        scratch_types=[
            pltpu.VMEM_SHARED((SPAD, D), jnp.float32),
            pltpu.VMEM((SCHUNK, D), jnp.float32),
            pltpu.VMEM((SUM_WAVE,), jnp.int32),
            pltpu.VMEM((SUM_WAVE,), jnp.int32),
            pltpu.SemaphoreType.DMA,
        ],
    )(src, dst, h1)


def _edge_body(src_hbm, dst_hbm, a_hbm, b_hbm, out_hbm,
               a_v, b_v, s_v, d_v, o_v):
    c = lax.axis_index("c")
    s = lax.axis_index("s")
    wid = s * NC + c
    base_e = wid * E_PER_W

    pltpu.sync_copy(a_hbm, a_v.at[pl.ds(0, N)])
    pltpu.sync_copy(b_hbm, b_v.at[pl.ds(0, N)])
    pltpu.sync_copy(src_hbm.at[pl.ds(base_e, E_PER_W)],
                    s_v.at[pl.ds(0, E_PER_W)])
    pltpu.sync_copy(dst_hbm.at[pl.ds(base_e, E_PER_W)],
                    d_v.at[pl.ds(0, E_PER_W)])

    def lp(g, _):
        sv = s_v[pl.ds(g * 16, 16)]
        dv = d_v[pl.ds(g * 16, 16)]
        for k in range(16):
            va = a_v[pl.ds(sv[k], 16)][0]
            vb = b_v[pl.ds(dv[k], 16)][0]
            o_v[pl.ds(g * 16 + k, 16)] = jnp.zeros((16,), jnp.float32) \
                + (va + vb)
        return 0
    lax.fori_loop(0, E_PER_W // 16, lp, 0)

    pltpu.sync_copy(o_v.at[pl.ds(0, E_PER_W)],
                    out_hbm.at[pl.ds(base_e, E_PER_W)])


def _edge_scores(src, dst, a, b):
    return pl.kernel(
        _edge_body,
        out_type=jax.ShapeDtypeStruct((E,), jnp.float32),
        mesh=_mesh(),
        scratch_types=[
            pltpu.VMEM((N + 16,), jnp.float32),
            pltpu.VMEM((N + 16,), jnp.float32),
            pltpu.VMEM((E_PER_W + 16,), jnp.int32),
            pltpu.VMEM((E_PER_W + 16,), jnp.int32),
            pltpu.VMEM((E_PER_W + 16,), jnp.float32),
        ],
    )(src, dst, a, b)


def _tc1_body(x_ref, wr_ref, br_ref, wp_ref, bp_ref, h_ref, m_ref):
    h = jnp.dot(x_ref[...], wr_ref[...],
                preferred_element_type=jnp.float32) + br_ref[...]
    h_ref[...] = h
    m = jnp.dot(h, wp_ref[...], preferred_element_type=jnp.float32)
    m_ref[...] = jnp.maximum(m + bp_ref[...], 0.0)


def _tc1(x, W_red, b_red, W_pool, b_pool):
    return pl.pallas_call(
        _tc1_body,
        grid=(GRID,),
        in_specs=[
            pl.BlockSpec((BLK, DIN), lambda i: (i, 0)),
            pl.BlockSpec((DIN, D), lambda i: (0, 0)),
            pl.BlockSpec((1, D), lambda i: (0, 0)),
            pl.BlockSpec((D, D), lambda i: (0, 0)),
            pl.BlockSpec((1, D), lambda i: (0, 0)),
        ],
        out_specs=[
            pl.BlockSpec((BLK, D), lambda i: (i, 0)),
            pl.BlockSpec((BLK, D), lambda i: (i, 0)),
        ],
        out_shape=[
            jax.ShapeDtypeStruct((N, D), jnp.float32),
            jax.ShapeDtypeStruct((N, D), jnp.float32),
        ],
    )(x, W_red, b_red.reshape(1, D), W_pool, b_pool.reshape(1, D))


def _tc2_body(h_ref, agg_ref, ws_ref, wn_ref, b_ref, h1_ref):
    r = (jnp.dot(h_ref[...], ws_ref[...], preferred_element_type=jnp.float32)
         + jnp.dot(agg_ref[...], wn_ref[...],
                   preferred_element_type=jnp.float32)
         + b_ref[...])
    n = jnp.sqrt(jnp.sum(r * r, axis=-1, keepdims=True))
    r = r / jnp.maximum(n, 1e-12)
    h1_ref[...] = jnp.maximum(r, 0.0)


def _tc2(h, agg, W_self1, W_neigh1, b1):
    return pl.pallas_call(
        _tc2_body,
        grid=(GRID,),
        in_specs=[
            pl.BlockSpec((BLK, D), lambda i: (i, 0)),
            pl.BlockSpec((BLK, D), lambda i: (i, 0)),
            pl.BlockSpec((D, D), lambda i: (0, 0)),
            pl.BlockSpec((D, D), lambda i: (0, 0)),
            pl.BlockSpec((1, D), lambda i: (0, 0)),
        ],
        out_specs=pl.BlockSpec((BLK, D), lambda i: (i, 0)),
        out_shape=jax.ShapeDtypeStruct((N, D), jnp.float32),
    )(h, agg, W_self1, W_neigh1, b1.reshape(1, D))


def _tc3_body(h1_ref, s0_ref, s1_ref, deg_ref, ws_ref, wn_ref, b_ref,
              we_ref, be_ref, ab_ref):
    mean = (s0_ref[...] + s1_ref[...]) / jnp.maximum(deg_ref[...], 1.0)
    h2 = (jnp.dot(h1_ref[...], ws_ref[...],
                  preferred_element_type=jnp.float32)
          + jnp.dot(mean, wn_ref[...], preferred_element_type=jnp.float32)
          + b_ref[...])
    h2 = jnp.maximum(h2, 0.0)
    ab = jnp.dot(h2, we_ref[...], preferred_element_type=jnp.float32)
    ab_ref[...] = ab + be_ref[...]


def _tc3(h1, s0, s1, deg, W_self2, W_neigh2, b2, we_p, be_p):
    return pl.pallas_call(
        _tc3_body,
        grid=(GRID,),
        in_specs=[
            pl.BlockSpec((BLK, D), lambda i: (i, 0)),
            pl.BlockSpec((BLK, D), lambda i: (i, 0)),
            pl.BlockSpec((BLK, D), lambda i: (i, 0)),
            pl.BlockSpec((BLK, 1), lambda i: (i, 0)),
            pl.BlockSpec((D, D), lambda i: (0, 0)),
            pl.BlockSpec((D, D), lambda i: (0, 0)),
            pl.BlockSpec((1, D), lambda i: (0, 0)),
            pl.BlockSpec((D, 8), lambda i: (0, 0)),
            pl.BlockSpec((1, 8), lambda i: (0, 0)),
        ],
        out_specs=pl.BlockSpec((BLK, 8), lambda i: (i, 0)),
        out_shape=jax.ShapeDtypeStruct((N, 8), jnp.float32),
    )(h1, s0, s1, deg, W_self2, W_neigh2, b2.reshape(1, D), we_p, be_p)


def kernel(x, edge_index, W_red, b_red, W_pool, b_pool, W_self1, W_neigh1,
           b1, W_self2, W_neigh2, b2, W_e, b_e):
    src = edge_index[0]
    dst = edge_index[1]

    h, m = _tc1(x, W_red, b_red, W_pool, b_pool)

    agg_t, deg_t = _segmax(src, dst, m)
    agg = agg_t.reshape(NPAD, D)[:N]
    deg = deg_t.reshape(NW, ACC_ROWS)[:, :RNG].reshape(NPAD)[:N].reshape(N, 1)

    h1 = _tc2(h, agg, W_self1, W_neigh1, b1)

    s_part = _segsum(src, dst, h1)
    s_full = s_part[:, :N, :]

    we_p = jnp.zeros((D, 8), jnp.float32)
    we_p = we_p.at[:, 0].set(W_e[:D, 0]).at[:, 1].set(W_e[D:, 0])
    be_p = jnp.zeros((1, 8), jnp.float32).at[0, 0].set(b_e[0])

    ab = _tc3(h1, s_full[0], s_full[1], deg, W_self2, W_neigh2, b2,
              we_p, be_p)
    a = ab[:, 0]
    b = ab[:, 1]

    out = _edge_scores(src, dst, a, b)
    return out.reshape(E, 1)
